# baseline (speedup 1.0000x reference)
import functools

import jax
import jax.numpy as jnp
from jax import lax
from jax.experimental import pallas as pl
from jax.experimental.pallas import tpu as pltpu
from jax.experimental.pallas import tpu_sc as plsc

NUM_CLASSES = 80
D = 256
NC, NS = 2, 16
NW = NC * NS
BSZ, N = 1024, 300
NB = 64
NN = 150
NSLOT = 6
NGRP = NN // NSLOT
NREP = 128


def _worker(table_hbm, idx_hbm, out_hbm, idx_v, buf, gsems, wsems):
    wid = lax.axis_index("s") * NC + lax.axis_index("c")
    n0 = (wid // 16) * NN
    b0 = pl.multiple_of((wid % 16) * NB, 8)
    base = pl.multiple_of(((wid % 16) * N + n0) * NB, 8)
    pltpu.sync_copy(idx_hbm.at[pl.ds(base, NN * NB)], idx_v)

    def start(t, j):
        off = pl.multiple_of(t * NB, 8)
        pltpu.async_copy(
            table_hbm.at[idx_v.at[pl.ds(off, NB)]], buf.at[j], gsems[j])

    def wait_gather(j):
        pltpu.make_async_copy(
            table_hbm.at[idx_v.at[pl.ds(0, NB)]], buf.at[j], gsems[j]).wait()

    def write(t, j):
        pltpu.async_copy(
            buf.at[j], out_hbm.at[n0 + t, pl.ds(b0, NB)], wsems[j])

    def wait_write(t, j):
        pltpu.make_async_copy(
            buf.at[j], out_hbm.at[n0 + t, pl.ds(b0, NB)], wsems[j]).wait()

    for j in range(3):
        start(j, j)

    for j in range(NSLOT):
        wait_gather(j)
        write(j, j)
        if j < 3:
            start(j + 3, j + 3)
        else:
            wait_write(j - 3, j - 3)
            start(j + 3, j - 3)

    def outer(g, carry):
        t0 = 6 * g
        for j in range(NSLOT):
            wait_gather(j)
            write(t0 + j, j)
            if j < 3:
                wait_write(t0 + j - 3, j + 3)
                start(t0 + j + 3, j + 3)
            else:
                wait_write(t0 + j, j - 3)
                start(t0 + j + 3, j - 3)
        return carry

    lax.fori_loop(1, NGRP - 1, outer, 0)

    t0 = NN - NSLOT
    for j in range(NSLOT):
        wait_gather(j)
        write(t0 + j, j)
        if j < 3:
            wait_write(t0 + j - 3, j + 3)
            start(t0 + j + 3, j + 3)
    for j in range(NSLOT):
        wait_write(t0 + j, j)


_sc_gather = functools.partial(
    pl.kernel,
    out_type=jax.ShapeDtypeStruct((N, BSZ, D), jnp.float32),
    mesh=plsc.VectorSubcoreMesh(core_axis_name="c", subcore_axis_name="s"),
    scratch_types=[
        pltpu.VMEM((NN * NB,), jnp.int32),
        pltpu.VMEM((NSLOT, NB, D), jnp.float32),
        [pltpu.SemaphoreType.DMA] * NSLOT,
        [pltpu.SemaphoreType.DMA] * NSLOT,
    ],
)(_worker)


def kernel(labels, label_embed_table):
    nrows = label_embed_table.shape[0]
    ones = jnp.ones((nrows, 1), label_embed_table.dtype)
    table = jnp.concatenate([label_embed_table, ones], axis=-1)
    table_rep = jnp.tile(table, (NREP, 1))
    rot = (jnp.arange(BSZ, dtype=jnp.int32) % NREP) * nrows
    idx = labels.T + rot[None, :]
    idxr = jnp.transpose(idx.reshape(N, 16, NB), (1, 0, 2)).reshape(-1)
    out = _sc_gather(table_rep, idxr)
    return jnp.transpose(out, (1, 0, 2))


# speedup vs baseline: 1.0193x; 1.0193x over previous
import functools

import jax
import jax.numpy as jnp
from jax import lax
from jax.experimental import pallas as pl
from jax.experimental.pallas import tpu as pltpu
from jax.experimental.pallas import tpu_sc as plsc

NUM_CLASSES = 80
D = 256
NC, NS = 2, 16
NW = NC * NS
BSZ, N = 1024, 300
NB = 64
NN = 150
NSLOT = 6
NGRP = NN // NSLOT
NREP = 64


def _worker(table_hbm, idx_hbm, out_hbm, idx_v, buf, gsems, wsems):
    wid = lax.axis_index("s") * NC + lax.axis_index("c")
    n0 = (wid // 16) * NN
    b0 = pl.multiple_of((wid % 16) * NB, 8)
    base = pl.multiple_of(((wid % 16) * N + n0) * NB, 8)
    pltpu.sync_copy(idx_hbm.at[pl.ds(base, NN * NB)], idx_v)

    def start(t, j):
        off = pl.multiple_of(t * NB, 8)
        pltpu.async_copy(
            table_hbm.at[idx_v.at[pl.ds(off, NB)]], buf.at[j], gsems[j])

    def wait_gather(j):
        pltpu.make_async_copy(
            table_hbm.at[idx_v.at[pl.ds(0, NB)]], buf.at[j], gsems[j]).wait()

    def write(t, j):
        pltpu.async_copy(
            buf.at[j], out_hbm.at[n0 + t, pl.ds(b0, NB)], wsems[j])

    def wait_write(t, j):
        pltpu.make_async_copy(
            buf.at[j], out_hbm.at[n0 + t, pl.ds(b0, NB)], wsems[j]).wait()

    for j in range(3):
        start(j, j)

    for j in range(NSLOT):
        wait_gather(j)
        write(j, j)
        if j < 3:
            start(j + 3, j + 3)
        else:
            wait_write(j - 3, j - 3)
            start(j + 3, j - 3)

    def outer(g, carry):
        t0 = 6 * g
        for j in range(NSLOT):
            wait_gather(j)
            write(t0 + j, j)
            if j < 3:
                wait_write(t0 + j - 3, j + 3)
                start(t0 + j + 3, j + 3)
            else:
                wait_write(t0 + j, j - 3)
                start(t0 + j + 3, j - 3)
        return carry

    lax.fori_loop(1, NGRP - 1, outer, 0)

    t0 = NN - NSLOT
    for j in range(NSLOT):
        wait_gather(j)
        write(t0 + j, j)
        if j < 3:
            wait_write(t0 + j - 3, j + 3)
            start(t0 + j + 3, j + 3)
    for j in range(NSLOT):
        wait_write(t0 + j, j)


_sc_gather = functools.partial(
    pl.kernel,
    out_type=jax.ShapeDtypeStruct((N, BSZ, D), jnp.float32),
    mesh=plsc.VectorSubcoreMesh(core_axis_name="c", subcore_axis_name="s"),
    scratch_types=[
        pltpu.VMEM((NN * NB,), jnp.int32),
        pltpu.VMEM((NSLOT, NB, D), jnp.float32),
        [pltpu.SemaphoreType.DMA] * NSLOT,
        [pltpu.SemaphoreType.DMA] * NSLOT,
    ],
)(_worker)


def kernel(labels, label_embed_table):
    nrows = label_embed_table.shape[0]
    ones = jnp.ones((nrows, 1), label_embed_table.dtype)
    table = jnp.concatenate([label_embed_table, ones], axis=-1)
    table_rep = jnp.tile(table, (NREP, 1))
    rot = (jnp.arange(BSZ, dtype=jnp.int32) % NREP) * nrows
    idx = labels.T + rot[None, :]
    idxr = jnp.transpose(idx.reshape(N, 16, NB), (1, 0, 2)).reshape(-1)
    out = _sc_gather(table_rep, idxr)
    return jnp.transpose(out, (1, 0, 2))
